# baseline (speedup 1.0000x reference)
import functools

import jax
import jax.numpy as jnp
from jax import lax
from jax.experimental import pallas as pl
from jax.experimental.pallas import tpu as pltpu
from jax.experimental.pallas import tpu_sc as plsc

D_EMBED = 64
NUM_CORES = 2
NUM_SUBCORES = 16
NUM_WORKERS = NUM_CORES * NUM_SUBCORES
NB = 8
NBUF = 4
SKEW = 2
HIST_PAD = 56
LANE_PAD = 128


@functools.partial(jax.jit, static_argnums=(2, 3))
def _gather(inp, table, batch, hist):
    rows_per_w = batch // NUM_WORKERS
    n_chunks = rows_per_w // NB
    n_groups = n_chunks // NBUF
    mesh = plsc.VectorSubcoreMesh(core_axis_name="c", subcore_axis_name="s")

    @functools.partial(
        pl.kernel,
        mesh=mesh,
        compiler_params=pltpu.CompilerParams(use_tc_tiling_on_sc=False),
        out_type=jax.ShapeDtypeStruct((batch, HIST_PAD, LANE_PAD), jnp.float32),
        scratch_types=(
            [pltpu.VMEM((NB, hist), jnp.int32) for _ in range(NBUF)]
            + [pltpu.VMEM((NB, hist, D_EMBED), jnp.float32) for _ in range(NBUF)]
            + [pltpu.SemaphoreType.DMA for _ in range(3 * NBUF)]
        ),
    )
    def k(idx_hbm, table_hbm, out_hbm, *bufs):
        idx_v = bufs[:NBUF]
        rows_v = bufs[NBUF : 2 * NBUF]
        si = bufs[2 * NBUF : 3 * NBUF]
        sg = bufs[3 * NBUF : 4 * NBUF]
        ss = bufs[4 * NBUF : 5 * NBUF]
        wid = lax.axis_index("s") * NUM_CORES + lax.axis_index("c")
        base = wid * rows_per_w

        def idx_copy(i, b):
            return pltpu.make_async_copy(
                idx_hbm.at[pl.ds(base + i * NB, NB)], idx_v[b], si[b]
            )

        def gather_copies(i, b):
            return [
                pltpu.make_async_copy(
                    table_hbm.at[idx_v[b].at[j]], rows_v[b].at[j], sg[b]
                )
                for j in range(NB)
            ]

        def store_copy(i, b):
            return pltpu.make_async_copy(
                rows_v[b],
                out_hbm.at[
                    pl.ds(base + i * NB, NB), pl.ds(0, hist), pl.ds(0, D_EMBED)
                ],
                ss[b],
            )

        def step(i, b, *, wait_store, prefetch, drain):
            if wait_store:
                store_copy(i - NBUF, b).wait()
            idx_copy(i, b).wait()
            for c in gather_copies(i, b):
                c.start()
            if drain:
                pb = (b - SKEW) % NBUF
                for c in gather_copies(i - SKEW, pb):
                    c.wait()
                store_copy(i - SKEW, pb).start()
                if prefetch:
                    idx_copy(i - SKEW + NBUF, pb).start()

        for b in range(NBUF):
            idx_copy(b, b).start()

        for b in range(NBUF):
            step(b, b, wait_store=False, prefetch=True, drain=(b >= SKEW))

        def body(g, _):
            i0 = g * NBUF
            for b in range(NBUF):
                step(i0 + b, b, wait_store=True, prefetch=True, drain=True)
            return 0

        lax.fori_loop(1, n_groups - 1, body, 0)

        i0 = (n_groups - 1) * NBUF
        for b in range(NBUF):
            step(i0 + b, b, wait_store=True, drain=True,
                 prefetch=(i0 + b - SKEW + NBUF < n_chunks))

        for i in range(n_chunks - SKEW, n_chunks):
            b = i % NBUF
            for c in gather_copies(i, b):
                c.wait()
            store_copy(i, b).start()
        for i in range(n_chunks - NBUF, n_chunks):
            store_copy(i, i % NBUF).wait()

    return k(inp, table)


def kernel(inp, table):
    batch, hist = inp.shape
    vocab = table.shape[0]
    t2 = jnp.pad(table[:, None, :], ((0, 0), (0, 1), (0, 0))).reshape(
        2 * vocab, D_EMBED
    )
    out_full = _gather(inp * 2, t2, batch, hist)
    return jax.lax.slice(out_full, (0, 0, 0), (batch, hist, D_EMBED))


# speedup vs baseline: 2.1051x; 2.1051x over previous
import functools

import jax
import jax.numpy as jnp
from jax import lax
from jax.experimental import pallas as pl
from jax.experimental.pallas import tpu as pltpu
from jax.experimental.pallas import tpu_sc as plsc

D_EMBED = 64
NUM_CORES = 2
NUM_SUBCORES = 16
NUM_WORKERS = NUM_CORES * NUM_SUBCORES
NB = 8
NBUF = 4
SKEW = 2
HIST_PAD = 56
LANE_PAD = 128


@functools.partial(jax.jit, static_argnums=(2, 3))
def _gather(inp, table, batch, hist):
    rows_per_w = batch // NUM_WORKERS
    n_chunks = rows_per_w // NB
    n_groups = n_chunks // NBUF
    mesh = plsc.VectorSubcoreMesh(core_axis_name="c", subcore_axis_name="s")

    @functools.partial(
        pl.kernel,
        mesh=mesh,
        compiler_params=pltpu.CompilerParams(use_tc_tiling_on_sc=False),
        out_type=jax.ShapeDtypeStruct((batch, HIST_PAD, LANE_PAD), jnp.float32),
        scratch_types=(
            [pltpu.VMEM((NB, hist), jnp.int32) for _ in range(NBUF)]
            + [pltpu.VMEM((NB, hist, D_EMBED), jnp.float32) for _ in range(NBUF)]
            + [pltpu.SemaphoreType.DMA for _ in range(3 * NBUF)]
        ),
    )
    def k(idx_hbm, table_hbm, out_hbm, *bufs):
        idx_v = bufs[:NBUF]
        rows_v = bufs[NBUF : 2 * NBUF]
        si = bufs[2 * NBUF : 3 * NBUF]
        sg = bufs[3 * NBUF : 4 * NBUF]
        ss = bufs[4 * NBUF : 5 * NBUF]
        wid = lax.axis_index("s") * NUM_CORES + lax.axis_index("c")
        base = wid * rows_per_w

        def idx_copy(i, b):
            return pltpu.make_async_copy(
                idx_hbm.at[pl.ds(base + i * NB, NB)], idx_v[b], si[b]
            )

        def gather_copies(i, b):
            return [
                pltpu.make_async_copy(
                    table_hbm.at[idx_v[b].at[j]], rows_v[b].at[j], sg[b]
                )
                for j in range(NB)
            ]

        def store_copy(i, b):
            return pltpu.make_async_copy(
                rows_v[b],
                out_hbm.at[
                    pl.ds(base + i * NB, NB), pl.ds(0, hist), pl.ds(0, D_EMBED)
                ],
                ss[b],
            )

        def step(i, b, *, wait_store, prefetch, drain):
            if wait_store:
                store_copy(i - NBUF, b).wait()
            idx_copy(i, b).wait()
            for c in gather_copies(i, b):
                c.start()
            if drain:
                pb = (b - SKEW) % NBUF
                for c in gather_copies(i - SKEW, pb):
                    c.wait()
                store_copy(i - SKEW, pb).start()
                if prefetch:
                    idx_copy(i - SKEW + NBUF, pb).start()

        for b in range(NBUF):
            idx_copy(b, b).start()

        for b in range(NBUF):
            step(b, b, wait_store=False, prefetch=True, drain=(b >= SKEW))

        def body(g, _):
            i0 = g * NBUF
            for b in range(NBUF):
                step(i0 + b, b, wait_store=True, prefetch=True, drain=True)
            return 0

        lax.fori_loop(1, n_groups - 1, body, 0)

        i0 = (n_groups - 1) * NBUF
        for b in range(NBUF):
            step(i0 + b, b, wait_store=True, drain=True,
                 prefetch=(i0 + b - SKEW + NBUF < n_chunks))

        for i in range(n_chunks - SKEW, n_chunks):
            b = i % NBUF
            for c in gather_copies(i, b):
                c.wait()
            store_copy(i, b).start()
        for i in range(n_chunks - NBUF, n_chunks):
            store_copy(i, i % NBUF).wait()

    return k(inp, table)


def kernel(inp, table):
    batch, hist = inp.shape
    vocab = table.shape[0]
    t2 = jnp.pad(table, ((0, 0), (0, LANE_PAD - D_EMBED))).reshape(
        2 * vocab, D_EMBED
    )
    out_full = _gather(inp * 2, t2, batch, hist)
    return jax.lax.slice(out_full, (0, 0, 0), (batch, hist, D_EMBED))
